# initial kernel scaffold (unmeasured)
import jax
import jax.numpy as jnp
from jax import lax
from jax.experimental import pallas as pl
from jax.experimental.pallas import tpu as pltpu


def kernel(
    x,
):
    def body(*refs):
        pass

    out_shape = jax.ShapeDtypeStruct(..., jnp.float32)
    return pl.pallas_call(body, out_shape=out_shape)(...)


# baseline (device time: 123606 ns/iter reference)
import jax
import jax.numpy as jnp
from jax import lax
from jax.experimental import pallas as pl
from jax.experimental.pallas import tpu as pltpu

M = 4096
N_GLOBAL = 2048
N_OUT = 1024
HALF = 512
CHUNKS = 1
M_CHUNK = M // CHUNKS


def kernel(x):
    x2 = x.reshape(M, N_GLOBAL).astype(jnp.bfloat16)

    def body(x_ref, out_ref, a_recv,
             a_send_sems, a_recv_sems, b_send_sems, b_recv_sems):
        my_x = lax.axis_index("x")
        my_y = lax.axis_index("y")
        peer_y = (my_x, 1 - my_y)
        peer_x = (1 - my_x, my_y)

        barrier = pltpu.get_barrier_semaphore()
        for nbr in (peer_y, peer_x):
            pl.semaphore_signal(barrier, inc=1, device_id=nbr,
                                device_id_type=pl.DeviceIdType.MESH)
        pl.semaphore_wait(barrier, 2)

        col_a_send = (1 - my_y) * N_OUT + my_x * HALF
        col_mine = my_y * N_OUT + my_x * HALF
        out_mine = my_x * HALF

        a_descs = []
        for c in range(CHUNKS):
            rows = pl.ds(c * M_CHUNK, M_CHUNK)
            d = pltpu.make_async_remote_copy(
                src_ref=x_ref.at[rows, pl.ds(col_a_send, HALF)],
                dst_ref=a_recv.at[rows, :],
                send_sem=a_send_sems.at[c],
                recv_sem=a_recv_sems.at[c],
                device_id=peer_y,
                device_id_type=pl.DeviceIdType.MESH,
            )
            d.start()
            a_descs.append(d)

        b_descs = []
        for c in range(CHUNKS):
            rows = pl.ds(c * M_CHUNK, M_CHUNK)
            a_descs[c].wait_recv()
            s = (x_ref[rows, pl.ds(col_mine, HALF)].astype(jnp.float32)
                 + a_recv[rows, :].astype(jnp.float32))
            out_ref[rows, pl.ds(out_mine, HALF)] = s.astype(jnp.bfloat16)
            d = pltpu.make_async_remote_copy(
                src_ref=out_ref.at[rows, pl.ds(out_mine, HALF)],
                dst_ref=out_ref.at[rows, pl.ds(out_mine, HALF)],
                send_sem=b_send_sems.at[c],
                recv_sem=b_recv_sems.at[c],
                device_id=peer_x,
                device_id_type=pl.DeviceIdType.MESH,
            )
            d.start()
            b_descs.append(d)

        for c in range(CHUNKS):
            b_descs[c].wait_recv()
        for c in range(CHUNKS):
            a_descs[c].wait_send()
            b_descs[c].wait_send()

    return pl.pallas_call(
        body,
        out_shape=jax.ShapeDtypeStruct((M, N_OUT), jnp.bfloat16),
        in_specs=[pl.BlockSpec(memory_space=pltpu.VMEM)],
        out_specs=pl.BlockSpec(memory_space=pltpu.VMEM),
        scratch_shapes=[
            pltpu.VMEM((M, HALF), jnp.bfloat16),
            pltpu.SemaphoreType.DMA((CHUNKS,)),
            pltpu.SemaphoreType.DMA((CHUNKS,)),
            pltpu.SemaphoreType.DMA((CHUNKS,)),
            pltpu.SemaphoreType.DMA((CHUNKS,)),
        ],
        compiler_params=pltpu.CompilerParams(collective_id=0),
    )(x2)


# device time: 83606 ns/iter; 1.4784x vs baseline; 1.4784x over previous
import jax
import jax.numpy as jnp
from jax import lax
from jax.experimental import pallas as pl
from jax.experimental.pallas import tpu as pltpu

M = 4096
N_GLOBAL = 2048
N_OUT = 1024
HALF = 512
CHUNKS = 8
M_CHUNK = M // CHUNKS


def kernel(x):
    x2 = x.reshape(M, N_GLOBAL).astype(jnp.bfloat16)

    def body(x_ref, out_ref, a_recv,
             a_send_sems, a_recv_sems, b_send_sems, b_recv_sems):
        my_x = lax.axis_index("x")
        my_y = lax.axis_index("y")
        peer_y = (my_x, 1 - my_y)
        peer_x = (1 - my_x, my_y)

        barrier = pltpu.get_barrier_semaphore()
        for nbr in (peer_y, peer_x):
            pl.semaphore_signal(barrier, inc=1, device_id=nbr,
                                device_id_type=pl.DeviceIdType.MESH)
        pl.semaphore_wait(barrier, 2)

        col_a_send = (1 - my_y) * N_OUT + my_x * HALF
        col_mine = my_y * N_OUT + my_x * HALF
        out_mine = my_x * HALF

        a_descs = []
        for c in range(CHUNKS):
            rows = pl.ds(c * M_CHUNK, M_CHUNK)
            d = pltpu.make_async_remote_copy(
                src_ref=x_ref.at[rows, pl.ds(col_a_send, HALF)],
                dst_ref=a_recv.at[rows, :],
                send_sem=a_send_sems.at[c],
                recv_sem=a_recv_sems.at[c],
                device_id=peer_y,
                device_id_type=pl.DeviceIdType.MESH,
            )
            d.start()
            a_descs.append(d)

        b_descs = []
        for c in range(CHUNKS):
            rows = pl.ds(c * M_CHUNK, M_CHUNK)
            a_descs[c].wait_recv()
            s = (x_ref[rows, pl.ds(col_mine, HALF)].astype(jnp.float32)
                 + a_recv[rows, :].astype(jnp.float32))
            out_ref[rows, pl.ds(out_mine, HALF)] = s.astype(jnp.bfloat16)
            d = pltpu.make_async_remote_copy(
                src_ref=out_ref.at[rows, pl.ds(out_mine, HALF)],
                dst_ref=out_ref.at[rows, pl.ds(out_mine, HALF)],
                send_sem=b_send_sems.at[c],
                recv_sem=b_recv_sems.at[c],
                device_id=peer_x,
                device_id_type=pl.DeviceIdType.MESH,
            )
            d.start()
            b_descs.append(d)

        for c in range(CHUNKS):
            b_descs[c].wait_recv()
        for c in range(CHUNKS):
            a_descs[c].wait_send()
            b_descs[c].wait_send()

    return pl.pallas_call(
        body,
        out_shape=jax.ShapeDtypeStruct((M, N_OUT), jnp.bfloat16),
        in_specs=[pl.BlockSpec(memory_space=pltpu.VMEM)],
        out_specs=pl.BlockSpec(memory_space=pltpu.VMEM),
        scratch_shapes=[
            pltpu.VMEM((M, HALF), jnp.bfloat16),
            pltpu.SemaphoreType.DMA((CHUNKS,)),
            pltpu.SemaphoreType.DMA((CHUNKS,)),
            pltpu.SemaphoreType.DMA((CHUNKS,)),
            pltpu.SemaphoreType.DMA((CHUNKS,)),
        ],
        compiler_params=pltpu.CompilerParams(collective_id=0),
    )(x2)


# device time: 67160 ns/iter; 1.8405x vs baseline; 1.2449x over previous
import jax
import jax.numpy as jnp
from jax import lax
from jax.experimental import pallas as pl
from jax.experimental.pallas import tpu as pltpu

M = 4096
N_GLOBAL = 2048
N_OUT = 1024
HALF = 512
CHUNKS = 16
M_CHUNK = M // CHUNKS


def kernel(x):
    my_x = lax.axis_index("x")
    my_y = lax.axis_index("y")
    x2d = x.reshape(M, N_GLOBAL)
    col_a_send = (1 - my_y) * N_OUT + my_x * HALF
    col_mine = my_y * N_OUT + my_x * HALF
    x2 = jnp.concatenate(
        [
            lax.dynamic_slice(x2d, (0, col_a_send), (M, HALF)),
            lax.dynamic_slice(x2d, (0, col_mine), (M, HALF)),
        ],
        axis=1,
    ).astype(jnp.bfloat16)

    def body(x_ref, out_ref, a_recv,
             a_send_sems, a_recv_sems, b_send_sems, b_recv_sems):
        my_x = lax.axis_index("x")
        my_y = lax.axis_index("y")
        peer_y = (my_x, 1 - my_y)
        peer_x = (1 - my_x, my_y)

        barrier = pltpu.get_barrier_semaphore()
        for nbr in (peer_y, peer_x):
            pl.semaphore_signal(barrier, inc=1, device_id=nbr,
                                device_id_type=pl.DeviceIdType.MESH)
        pl.semaphore_wait(barrier, 2)

        out_mine = my_x * HALF

        a_descs = []
        for c in range(CHUNKS):
            rows = pl.ds(c * M_CHUNK, M_CHUNK)
            d = pltpu.make_async_remote_copy(
                src_ref=x_ref.at[rows, pl.ds(0, HALF)],
                dst_ref=a_recv.at[rows, :],
                send_sem=a_send_sems.at[c],
                recv_sem=a_recv_sems.at[c],
                device_id=peer_y,
                device_id_type=pl.DeviceIdType.MESH,
            )
            d.start()
            a_descs.append(d)

        b_descs = []
        for c in range(CHUNKS):
            rows = pl.ds(c * M_CHUNK, M_CHUNK)
            a_descs[c].wait_recv()
            s = (x_ref[rows, pl.ds(HALF, HALF)].astype(jnp.float32)
                 + a_recv[rows, :].astype(jnp.float32))
            out_ref[rows, pl.ds(out_mine, HALF)] = s.astype(jnp.bfloat16)
            d = pltpu.make_async_remote_copy(
                src_ref=out_ref.at[rows, pl.ds(out_mine, HALF)],
                dst_ref=out_ref.at[rows, pl.ds(out_mine, HALF)],
                send_sem=b_send_sems.at[c],
                recv_sem=b_recv_sems.at[c],
                device_id=peer_x,
                device_id_type=pl.DeviceIdType.MESH,
            )
            d.start()
            b_descs.append(d)

        for c in range(CHUNKS):
            b_descs[c].wait_recv()
        for c in range(CHUNKS):
            a_descs[c].wait_send()
            b_descs[c].wait_send()

    return pl.pallas_call(
        body,
        out_shape=jax.ShapeDtypeStruct((M, N_OUT), jnp.bfloat16),
        in_specs=[pl.BlockSpec(memory_space=pltpu.VMEM)],
        out_specs=pl.BlockSpec(memory_space=pltpu.VMEM),
        scratch_shapes=[
            pltpu.VMEM((M, HALF), jnp.bfloat16),
            pltpu.SemaphoreType.DMA((CHUNKS,)),
            pltpu.SemaphoreType.DMA((CHUNKS,)),
            pltpu.SemaphoreType.DMA((CHUNKS,)),
            pltpu.SemaphoreType.DMA((CHUNKS,)),
        ],
        compiler_params=pltpu.CompilerParams(collective_id=0),
    )(x2)
